# baseline (speedup 1.0000x reference)
import jax
import jax.numpy as jnp
from jax.experimental import pallas as pl

IN_F = 784
OUT_F = 128
NK = 20
NB = 512
BB = 8

_MINT = -2147483648
_M7F = 2147483647


def _decode(k):
    s = jnp.where(k < 0, jnp.bitwise_xor(k, _M7F), k)
    return jax.lax.bitcast_convert_type(s, jnp.float32)


def _encode(v):
    s = jax.lax.bitcast_convert_type(v, jnp.int32)
    return jnp.where(s < 0, jnp.bitwise_xor(s, _M7F), s)


def _mid_ceil(kl, kh):
    ql = jnp.bitwise_xor(kl, _MINT)
    qh = jnp.bitwise_xor(kh, _MINT)
    x = jnp.bitwise_xor(ql, qh)
    mid_q = (jnp.bitwise_and(ql, qh)
             + jax.lax.shift_right_logical(x, 1)
             + jnp.bitwise_and(x, 1))
    return jnp.bitwise_xor(mid_q, _MINT)


def _mini_radix(a, kf):
    def step(i, p):
        bit = jax.lax.shift_left(jnp.int32(1), jnp.int32(31) - i)
        c = jnp.bitwise_or(p, bit)
        t = _decode(jnp.bitwise_xor(c, _MINT))[..., None]
        cnt = jnp.sum(jnp.where(a >= t, 1.0, 0.0), axis=-1)
        return jnp.where(cnt >= kf, c, p)
    p0 = jnp.zeros(a.shape[:-1], jnp.int32)
    p = jax.lax.fori_loop(0, 32, step, p0)
    return jnp.bitwise_xor(p, _MINT)


def _body(x_ref, w_ref, b_ref, o_ref):
    xb = x_ref[...]
    w = w_ref[...]
    bias = b_ref[...]

    dot = jax.lax.dot_general(
        xb, w, dimension_numbers=(((1,), (1,)), ((), ())),
        preferred_element_type=jnp.float32)

    temp = xb[:, None, :] * w[None, :, :]
    kf = jnp.float32(NK)
    bb = temp.shape[0]

    ninf = jnp.float32(jnp.finfo(jnp.float32).min)
    pinf = jnp.float32(jnp.finfo(jnp.float32).max)
    padmax = jnp.full((bb, OUT_F, 1024 - IN_F), ninf, jnp.float32)
    gmax = jnp.concatenate([temp, padmax], axis=-1)
    gneg = jnp.concatenate([-temp, padmax], axis=-1)
    for half in (512, 256, 128, 64):
        gmax = jnp.maximum(gmax[..., :half], gmax[..., half:])
        gneg = jnp.maximum(gneg[..., :half], gneg[..., half:])

    kl_hi = _mini_radix(gmax, kf)
    kl_lo = _mini_radix(gneg, kf)
    kh_hi = _encode(jnp.max(gmax, axis=-1))
    kh_lo = _encode(jnp.max(gneg, axis=-1))

    def count_hi(t):
        return jnp.sum(jnp.where(temp >= t[:, :, None], 1.0, 0.0), axis=-1)

    def count_lo(t):
        return jnp.sum(jnp.where(temp <= (-t)[:, :, None], 1.0, 0.0), axis=-1)

    c_hi = count_hi(_decode(kl_hi))
    c_lo = count_lo(_decode(kl_lo))

    def active(c, kl, kh):
        return jnp.logical_and(c != kf, kh > kl)

    def cond(state):
        kl_h, kh_h, c_h, kl_l, kh_l, c_l = state
        return jnp.logical_or(jnp.any(active(c_h, kl_h, kh_h)),
                              jnp.any(active(c_l, kl_l, kh_l)))

    def body(state):
        kl_h, kh_h, c_h, kl_l, kh_l, c_l = state

        act = active(c_h, kl_h, kh_h)
        mid = _mid_ceil(kl_h, kh_h)
        cnt = count_hi(_decode(mid))
        ge = cnt >= kf
        kl_h = jnp.where(jnp.logical_and(act, ge), mid, kl_h)
        c_h = jnp.where(jnp.logical_and(act, ge), cnt, c_h)
        kh_h = jnp.where(jnp.logical_and(act, jnp.logical_not(ge)), mid - 1, kh_h)

        act = active(c_l, kl_l, kh_l)
        mid = _mid_ceil(kl_l, kh_l)
        cnt = count_lo(_decode(mid))
        ge = cnt >= kf
        kl_l = jnp.where(jnp.logical_and(act, ge), mid, kl_l)
        c_l = jnp.where(jnp.logical_and(act, ge), cnt, c_l)
        kh_l = jnp.where(jnp.logical_and(act, jnp.logical_not(ge)), mid - 1, kh_l)

        return kl_h, kh_h, c_h, kl_l, kh_l, c_l

    kl_hi, kh_hi, c_hi, kl_lo, kh_lo, c_lo = jax.lax.while_loop(
        cond, body, (kl_hi, kh_hi, c_hi, kl_lo, kh_lo, c_lo))

    tl_hi = _decode(kl_hi)
    tl_lo = _decode(kl_lo)
    min_hi = jnp.min(jnp.where(temp >= tl_hi[:, :, None], temp, pinf), axis=-1)
    min_lo = jnp.min(jnp.where(temp <= (-tl_lo)[:, :, None], -temp, pinf), axis=-1)
    t_hi = jnp.where(c_hi == kf, min_hi, tl_hi)
    t_lo = jnp.where(c_lo == kf, min_lo, tl_lo)

    sum_hi = jnp.sum(jnp.maximum(temp - t_hi[:, :, None], 0.0), axis=-1)
    sum_lo = jnp.sum(jnp.maximum(-temp - t_lo[:, :, None], 0.0), axis=-1)

    o_ref[...] = dot - (kf * t_hi + sum_hi) + (kf * t_lo + sum_lo) + bias


def kernel(x, W, b):
    b2 = b.reshape(1, OUT_F)
    return pl.pallas_call(
        _body,
        grid=(NB // BB,),
        in_specs=[
            pl.BlockSpec((BB, IN_F), lambda i: (i, 0)),
            pl.BlockSpec((OUT_F, IN_F), lambda i: (0, 0)),
            pl.BlockSpec((1, OUT_F), lambda i: (0, 0)),
        ],
        out_specs=pl.BlockSpec((BB, OUT_F), lambda i: (i, 0)),
        out_shape=jax.ShapeDtypeStruct((NB, OUT_F), jnp.float32),
    )(x, W, b2)
